# Initial kernel scaffold; baseline (speedup 1.0000x reference)
import jax
import jax.numpy as jnp
from jax.experimental import pallas as pl


def kernel(X, vq_codebook, emb_table):
    raise NotImplementedError("write your pallas kernel here")


# speedup vs baseline: 5.8091x; 5.8091x over previous
import functools

import jax
import jax.numpy as jnp
from jax import lax
from jax.experimental import pallas as pl
from jax.experimental.pallas import tpu as pltpu
from jax.experimental.pallas import tpu_sc as plsc

_B = 8
_T = 98432
_K = 256
_STRIDE = 128
_NUM_EMB = 1024
_D = 256
_N = 768
_FLAT = _B * _N


def _tc_encode_body(x_ref, cbt_ref, idx_ref):
    x = x_ref[0]
    a = x[0:_N, :]
    b = x[1:_N + 1, :]
    tokens = jnp.concatenate([a, b], axis=1)
    cbt = cbt_ref[...]
    cnorm = jnp.sum(cbt * cbt, axis=0, keepdims=True)
    scores = cnorm - 2.0 * jnp.dot(tokens, cbt, preferred_element_type=jnp.float32)
    m = jnp.min(scores, axis=1, keepdims=True)
    cols = lax.broadcasted_iota(jnp.int32, scores.shape, 1)
    idx = jnp.min(jnp.where(scores <= m, cols, _NUM_EMB), axis=1)
    idx_ref[0, 0, :] = idx


def _tc_encode(xr, cbt):
    return pl.pallas_call(
        _tc_encode_body,
        grid=(_B,),
        in_specs=[
            pl.BlockSpec((1, _N + 1, _STRIDE), lambda i: (i, 0, 0)),
            pl.BlockSpec((_K, _NUM_EMB), lambda i: (0, 0)),
        ],
        out_specs=pl.BlockSpec((1, 1, _N), lambda i: (i, 0, 0)),
        out_shape=jax.ShapeDtypeStruct((_B, 1, _N), jnp.int32),
    )(xr, cbt)


def _make_sc_gather():
    info = plsc.get_sparse_core_info()
    nc, ns = info.num_cores, info.num_subcores
    nw = nc * ns
    rows_per_w = _FLAT // nw
    half = rows_per_w // 2
    mesh = plsc.VectorSubcoreMesh(core_axis_name="c", subcore_axis_name="s")

    @functools.partial(
        pl.kernel,
        mesh=mesh,
        out_type=jax.ShapeDtypeStruct((_FLAT, _D), jnp.float32),
        scratch_types=[
            pltpu.VMEM((half,), jnp.int32),
            pltpu.VMEM((half,), jnp.int32),
            pltpu.VMEM((rows_per_w, _D), jnp.float32),
            pltpu.VMEM((rows_per_w, _D), jnp.float32),
            pltpu.SemaphoreType.DMA,
        ],
    )
    def sc_gather(emb_hbm, idx_hbm, pe_hbm, out_hbm,
                  idx_v0, idx_v1, rows_v, pe_v, sem):
        w = lax.axis_index("s") * nc + lax.axis_index("c")
        pltpu.sync_copy(idx_hbm.at[2 * w], idx_v0)
        pltpu.sync_copy(idx_hbm.at[2 * w + 1], idx_v1)
        cp0 = pltpu.async_copy(emb_hbm.at[idx_v0], rows_v.at[pl.ds(0, half)], sem)
        cp1 = pltpu.async_copy(emb_hbm.at[idx_v1], rows_v.at[pl.ds(half, half)], sem)
        pltpu.sync_copy(pe_hbm.at[pl.ds((w % (_N // rows_per_w)) * rows_per_w,
                                        rows_per_w)], pe_v)
        cp0.wait()
        cp1.wait()

        def add_row(r, carry):
            for c in range(_D // 16):
                sl = pl.ds(c * 16, 16)
                rows_v[r, sl] = rows_v[r, sl] + pe_v[r, sl]
            return carry

        lax.fori_loop(0, rows_per_w, add_row, 0)
        pltpu.sync_copy(rows_v, out_hbm.at[pl.ds(w * rows_per_w, rows_per_w)])

    return sc_gather, half


def _positional_table():
    i = jnp.arange(_D // 2, dtype=jnp.float32)
    t = 1.0 / (10000.0 ** (2.0 * i / _D))
    pos = jnp.arange(_N, dtype=jnp.float32)[:, None] * _STRIDE
    s = jnp.sin(pos * t[None, :])
    c = jnp.cos(pos * t[None, :])
    pe = jnp.stack([s, c], axis=1)
    return jnp.transpose(pe, (0, 2, 1)).reshape(_N, _D)


def kernel(X, vq_codebook, emb_table):
    xr = X.reshape(_B, _N + 1, _STRIDE)
    cbt = vq_codebook.T
    idx3 = _tc_encode(xr, cbt)
    sc_gather, half = _make_sc_gather()
    idx2 = idx3.reshape(_FLAT // half, half)
    pe = _positional_table()
    out_flat = sc_gather(emb_table, idx2, pe)
    return out_flat.reshape(_B, _N, _D)


# speedup vs baseline: 6.8063x; 1.1717x over previous
import functools

import jax
import jax.numpy as jnp
from jax import lax
from jax.experimental import pallas as pl
from jax.experimental.pallas import tpu as pltpu
from jax.experimental.pallas import tpu_sc as plsc

_B = 8
_T = 98432
_K = 256
_STRIDE = 128
_NUM_EMB = 1024
_D = 256
_N = 768
_FLAT = _B * _N


def _tc_encode_body(x_ref, cb_ref, idx_ref):
    x = x_ref[0]
    a = x[0:_N, :]
    b = x[1:_N + 1, :]
    tokens = jnp.concatenate([a, b], axis=1)
    cb = cb_ref[...]
    cnorm = jnp.sum(cb * cb, axis=1, keepdims=True)
    prod = lax.dot_general(cb, tokens, (((1,), (1,)), ((), ())),
                           preferred_element_type=jnp.float32)
    scores = cnorm - 2.0 * prod
    m = jnp.min(scores, axis=0, keepdims=True)
    rows = lax.broadcasted_iota(jnp.int32, scores.shape, 0).astype(jnp.float32)
    idxf = jnp.min(jnp.where(scores <= m, rows, float(_NUM_EMB)), axis=0)
    idx_ref[0, 0, :] = idxf.astype(jnp.int32)


def _tc_encode(xr, cb):
    return pl.pallas_call(
        _tc_encode_body,
        grid=(_B,),
        in_specs=[
            pl.BlockSpec((1, _N + 1, _STRIDE), lambda i: (i, 0, 0)),
            pl.BlockSpec((_NUM_EMB, _K), lambda i: (0, 0)),
        ],
        out_specs=pl.BlockSpec((1, 1, _N), lambda i: (i, 0, 0)),
        out_shape=jax.ShapeDtypeStruct((_B, 1, _N), jnp.int32),
    )(xr, cb)


def _make_sc_gather():
    info = plsc.get_sparse_core_info()
    nc, ns = info.num_cores, info.num_subcores
    nw = nc * ns
    rows_per_w = _FLAT // nw
    half = rows_per_w // 2
    mesh = plsc.VectorSubcoreMesh(core_axis_name="c", subcore_axis_name="s")

    @functools.partial(
        pl.kernel,
        mesh=mesh,
        out_type=jax.ShapeDtypeStruct((_FLAT, _D), jnp.float32),
        scratch_types=[
            pltpu.VMEM((half,), jnp.int32),
            pltpu.VMEM((half,), jnp.int32),
            pltpu.VMEM((rows_per_w, _D), jnp.float32),
            pltpu.VMEM((rows_per_w, _D), jnp.float32),
            pltpu.SemaphoreType.DMA,
        ],
    )
    def sc_gather(emb_hbm, idx_hbm, pe_hbm, out_hbm,
                  idx_v0, idx_v1, rows_v, pe_v, sem):
        w = lax.axis_index("s") * nc + lax.axis_index("c")
        pltpu.sync_copy(idx_hbm.at[2 * w], idx_v0)
        pltpu.sync_copy(idx_hbm.at[2 * w + 1], idx_v1)
        cp0 = pltpu.async_copy(emb_hbm.at[idx_v0], rows_v.at[pl.ds(0, half)], sem)
        cp1 = pltpu.async_copy(emb_hbm.at[idx_v1], rows_v.at[pl.ds(half, half)], sem)
        pltpu.sync_copy(pe_hbm.at[pl.ds((w % (_N // rows_per_w)) * rows_per_w,
                                        rows_per_w)], pe_v)
        cp0.wait()
        cp1.wait()

        def add_row(r, carry):
            for c in range(_D // 16):
                sl = pl.ds(c * 16, 16)
                rows_v[r, sl] = rows_v[r, sl] + pe_v[r, sl]
            return carry

        lax.fori_loop(0, rows_per_w, add_row, 0)
        pltpu.sync_copy(rows_v, out_hbm.at[pl.ds(w * rows_per_w, rows_per_w)])

    return sc_gather, half


def _positional_table():
    i = jnp.arange(_D // 2, dtype=jnp.float32)
    t = 1.0 / (10000.0 ** (2.0 * i / _D))
    pos = jnp.arange(_N, dtype=jnp.float32)[:, None] * _STRIDE
    s = jnp.sin(pos * t[None, :])
    c = jnp.cos(pos * t[None, :])
    pe = jnp.stack([s, c], axis=1)
    return jnp.transpose(pe, (0, 2, 1)).reshape(_N, _D)


def kernel(X, vq_codebook, emb_table):
    xr = X.reshape(_B, _N + 1, _STRIDE)
    idx3 = _tc_encode(xr, vq_codebook)
    sc_gather, half = _make_sc_gather()
    idx2 = idx3.reshape(_FLAT // half, half)
    pe = _positional_table()
    out_flat = sc_gather(emb_table, idx2, pe)
    return out_flat.reshape(_B, _N, _D)


# speedup vs baseline: 6.8839x; 1.0114x over previous
import functools

import jax
import jax.numpy as jnp
from jax import lax
from jax.experimental import pallas as pl
from jax.experimental.pallas import tpu as pltpu
from jax.experimental.pallas import tpu_sc as plsc

_B = 8
_T = 98432
_K = 256
_STRIDE = 128
_NUM_EMB = 1024
_D = 256
_N = 768
_FLAT = _B * _N


def _tc_encode_body(x_ref, cb_ref, idx_ref):
    x = x_ref[0]
    a = x[0:_N, :]
    b = x[1:_N + 1, :]
    tokens = jnp.concatenate([a, b], axis=1)
    cb = cb_ref[...]
    cnorm = jnp.sum(cb * cb, axis=1, keepdims=True)
    prod = lax.dot_general(cb, tokens, (((1,), (1,)), ((), ())),
                           preferred_element_type=jnp.float32)
    scores = cnorm - 2.0 * prod
    m = jnp.min(scores, axis=0, keepdims=True)
    rows = lax.broadcasted_iota(jnp.int32, scores.shape, 0).astype(jnp.float32)
    idxf = jnp.min(jnp.where(scores <= m, rows, float(_NUM_EMB)), axis=0)
    idx_ref[0, 0, :] = idxf.astype(jnp.int32)


def _tc_encode(xr, cb):
    return pl.pallas_call(
        _tc_encode_body,
        grid=(_B,),
        in_specs=[
            pl.BlockSpec((1, _N + 1, _STRIDE), lambda i: (i, 0, 0)),
            pl.BlockSpec((_NUM_EMB, _K), lambda i: (0, 0)),
        ],
        out_specs=pl.BlockSpec((1, 1, _N), lambda i: (i, 0, 0)),
        out_shape=jax.ShapeDtypeStruct((_B, 1, _N), jnp.int32),
    )(xr, cb)


def _make_sc_gather():
    info = plsc.get_sparse_core_info()
    nc, ns = info.num_cores, info.num_subcores
    nw = nc * ns
    rows_per_w = _FLAT // nw
    half = rows_per_w // 2
    mesh = plsc.VectorSubcoreMesh(core_axis_name="c", subcore_axis_name="s")

    @functools.partial(
        pl.kernel,
        mesh=mesh,
        out_type=jax.ShapeDtypeStruct((_FLAT, _D), jnp.float32),
        scratch_types=[
            pltpu.VMEM((half,), jnp.int32),
            pltpu.VMEM((half,), jnp.int32),
            pltpu.VMEM((rows_per_w, _D), jnp.float32),
            pltpu.VMEM((rows_per_w, _D), jnp.float32),
            pltpu.SemaphoreType.DMA,
            pltpu.SemaphoreType.DMA,
            pltpu.SemaphoreType.DMA,
            pltpu.SemaphoreType.DMA,
        ],
    )
    def sc_gather(emb_hbm, idx_hbm, pe_hbm, out_hbm,
                  idx_v0, idx_v1, rows_v, pe_v, sem0, sem1, sem_pe, sem_out):
        w = lax.axis_index("s") * nc + lax.axis_index("c")
        pltpu.sync_copy(idx_hbm.at[2 * w], idx_v0)
        pltpu.sync_copy(idx_hbm.at[2 * w + 1], idx_v1)
        cp0 = pltpu.async_copy(emb_hbm.at[idx_v0], rows_v.at[pl.ds(0, half)], sem0)
        cp1 = pltpu.async_copy(emb_hbm.at[idx_v1], rows_v.at[pl.ds(half, half)], sem1)
        cpe = pltpu.async_copy(
            pe_hbm.at[pl.ds((w % (_N // rows_per_w)) * rows_per_w, rows_per_w)],
            pe_v, sem_pe)
        cpe.wait()
        cp0.wait()

        def add_half(base):
            @plsc.parallel_loop(base, base + half, 1, unroll=2)
            def _(r):
                for c in range(_D // 16):
                    sl = pl.ds(c * 16, 16)
                    rows_v[r, sl] = rows_v[r, sl] + pe_v[r, sl]

        add_half(0)
        co0 = pltpu.async_copy(rows_v.at[pl.ds(0, half)],
                               out_hbm.at[pl.ds(w * rows_per_w, half)], sem_out)
        cp1.wait()
        add_half(half)
        co1 = pltpu.async_copy(rows_v.at[pl.ds(half, half)],
                               out_hbm.at[pl.ds(w * rows_per_w + half, half)],
                               sem_out)
        co0.wait()
        co1.wait()

    return sc_gather, half


def _positional_table():
    i = jnp.arange(_D // 2, dtype=jnp.float32)
    t = 1.0 / (10000.0 ** (2.0 * i / _D))
    pos = jnp.arange(_N, dtype=jnp.float32)[:, None] * _STRIDE
    s = jnp.sin(pos * t[None, :])
    c = jnp.cos(pos * t[None, :])
    pe = jnp.stack([s, c], axis=1)
    return jnp.transpose(pe, (0, 2, 1)).reshape(_N, _D)


def kernel(X, vq_codebook, emb_table):
    xr = X.reshape(_B, _N + 1, _STRIDE)
    idx3 = _tc_encode(xr, vq_codebook)
    sc_gather, half = _make_sc_gather()
    idx2 = idx3.reshape(_FLAT // half, half)
    pe = _positional_table()
    out_flat = sc_gather(emb_table, idx2, pe)
    return out_flat.reshape(_B, _N, _D)


# speedup vs baseline: 7.2946x; 1.0597x over previous
import functools

import jax
import jax.numpy as jnp
import numpy as np
from jax import lax
from jax.experimental import pallas as pl
from jax.experimental.pallas import tpu as pltpu
from jax.experimental.pallas import tpu_sc as plsc

_B = 8
_T = 98432
_K = 256
_STRIDE = 128
_NUM_EMB = 1024
_D = 256
_N = 768
_FLAT = _B * _N


def _tc_encode_body(x_ref, cb_ref, idx_ref):
    x = x_ref[0]
    a = x[0:_N, :]
    b = x[1:_N + 1, :]
    tokens = jnp.concatenate([a, b], axis=1)
    cb = cb_ref[...]
    cnorm = jnp.sum(cb * cb, axis=1, keepdims=True)
    prod = lax.dot_general(cb, tokens, (((1,), (1,)), ((), ())),
                           preferred_element_type=jnp.float32)
    scores = cnorm - 2.0 * prod
    m = jnp.min(scores, axis=0, keepdims=True)
    rows = lax.broadcasted_iota(jnp.int32, scores.shape, 0).astype(jnp.float32)
    idxf = jnp.min(jnp.where(scores <= m, rows, float(_NUM_EMB)), axis=0)
    idx_ref[0, 0, :] = idxf.astype(jnp.int32)


def _tc_encode(xr, cb):
    return pl.pallas_call(
        _tc_encode_body,
        grid=(_B,),
        in_specs=[
            pl.BlockSpec((1, _N + 1, _STRIDE), lambda i: (i, 0, 0)),
            pl.BlockSpec((_NUM_EMB, _K), lambda i: (0, 0)),
        ],
        out_specs=pl.BlockSpec((1, 1, _N), lambda i: (i, 0, 0)),
        out_shape=jax.ShapeDtypeStruct((_B, 1, _N), jnp.int32),
    )(xr, cb)


_NCHUNK = 4


def _make_sc_gather():
    info = plsc.get_sparse_core_info()
    nc, ns = info.num_cores, info.num_subcores
    nw = nc * ns
    rows_per_w = _FLAT // nw
    chunk = rows_per_w // _NCHUNK
    mesh = plsc.VectorSubcoreMesh(core_axis_name="c", subcore_axis_name="s")

    @functools.partial(
        pl.kernel,
        mesh=mesh,
        out_type=jax.ShapeDtypeStruct((_FLAT, _D), jnp.float32),
        scratch_types=[
            pltpu.VMEM((rows_per_w,), jnp.int32),
            pltpu.VMEM((rows_per_w, _D), jnp.float32),
            pltpu.VMEM((rows_per_w, _D), jnp.float32),
            [pltpu.SemaphoreType.DMA] * _NCHUNK,
            pltpu.SemaphoreType.DMA,
            pltpu.SemaphoreType.DMA,
        ],
    )
    def sc_gather(emb_hbm, idx_hbm, pe_hbm, out_hbm,
                  idx_v, rows_v, pe_v, gsems, sem_pe, sem_out):
        w = lax.axis_index("s") * nc + lax.axis_index("c")
        base = w * rows_per_w
        pltpu.sync_copy(idx_hbm.at[pl.ds(base, rows_per_w)], idx_v)
        gathers = []
        for j in range(_NCHUNK):
            sl = pl.ds(j * chunk, chunk)
            gathers.append(pltpu.async_copy(
                emb_hbm.at[idx_v.at[sl]], rows_v.at[sl], gsems[j]))
        cpe = pltpu.async_copy(
            pe_hbm.at[pl.ds((w % (_N // rows_per_w)) * rows_per_w, rows_per_w)],
            pe_v, sem_pe)
        cpe.wait()
        writebacks = []
        for j in range(_NCHUNK):
            gathers[j].wait()

            @plsc.parallel_loop(j * chunk, (j + 1) * chunk, 1, unroll=2)
            def _(r):
                for c in range(_D // 16):
                    sl16 = pl.ds(c * 16, 16)
                    rows_v[r, sl16] = rows_v[r, sl16] + pe_v[r, sl16]

            sl = pl.ds(j * chunk, chunk)
            writebacks.append(pltpu.async_copy(
                rows_v.at[sl], out_hbm.at[pl.ds(base + j * chunk, chunk)],
                sem_out))
        for co in writebacks:
            co.wait()

    return sc_gather


def _positional_table():
    i = np.arange(_D // 2, dtype=np.float64)
    t = 1.0 / (10000.0 ** (2.0 * i / _D))
    pos = np.arange(_N, dtype=np.float64)[:, None] * _STRIDE
    s = np.sin(pos * t[None, :])
    c = np.cos(pos * t[None, :])
    pe = np.stack([s, c], axis=1)
    pe = np.transpose(pe, (0, 2, 1)).reshape(_N, _D)
    return jnp.asarray(pe.astype(np.float32))


def kernel(X, vq_codebook, emb_table):
    xr = X.reshape(_B, _N + 1, _STRIDE)
    idx3 = _tc_encode(xr, vq_codebook)
    sc_gather = _make_sc_gather()
    pe = _positional_table()
    out_flat = sc_gather(emb_table, idx3.reshape(_FLAT), pe)
    return out_flat.reshape(_B, _N, _D)
